# baseline (speedup 1.0000x reference)
import functools

import jax
import jax.numpy as jnp
from jax import lax
from jax.experimental import pallas as pl
from jax.experimental.pallas import tpu as pltpu
from jax.experimental.pallas import tpu_sc as plsc

_NUM_CORES = 2
_NUM_SUBCORES = 16
_NUM_WORKERS = _NUM_CORES * _NUM_SUBCORES
_L = 16


def _make_gather(n_out: int, n_rows: int, d: int, nbuf: int):
  rpt = n_rows // _NUM_WORKERS
  rpc = n_rows // _NUM_CORES
  _C = 128
  pad_cap = n_out + _NUM_SUBCORES * _C
  mesh = plsc.VectorSubcoreMesh(core_axis_name="c", subcore_axis_name="s")

  @functools.partial(
      pl.kernel,
      out_type=[
          jax.ShapeDtypeStruct((n_out, d), jnp.float32),
          jax.ShapeDtypeStruct((_NUM_CORES, _NUM_SUBCORES, _C), jnp.int32),
          jax.ShapeDtypeStruct((_NUM_CORES, pad_cap), jnp.int32),
      ],
      mesh=mesh,
      compiler_params=pltpu.CompilerParams(needs_layout_passes=False),
      scratch_types=[
          pltpu.VMEM((n_out,), jnp.int32),
          pltpu.VMEM((n_out,), jnp.int32),
          pltpu.VMEM((n_out,), jnp.int32),
          pltpu.VMEM((n_out,), jnp.int32),
          pltpu.VMEM((pad_cap,), jnp.int32),
          pltpu.VMEM((rpt,), jnp.int32),
          pltpu.VMEM((rpt,), jnp.int32),
          pltpu.VMEM((rpt,), jnp.int32),
          pltpu.VMEM((_C,), jnp.int32),
          pltpu.VMEM((_NUM_SUBCORES, _C), jnp.int32),
          pltpu.VMEM((rpc,), jnp.int32),
          [pltpu.VMEM((1, d), jnp.float32) for _ in range(nbuf)],
          [pltpu.SemaphoreType.DMA for _ in range(nbuf)],
          [pltpu.SemaphoreType.DMA for _ in range(nbuf)],
      ],
  )
  def gather_kernel(idx_hbm, table_hbm, out_hbm, hist_hbm, grouped_hbm,
                    idx_full, wl_pos, wl_row, grouped, grouped_all, hist,
                    starts, cursor, hist_pub, hist_all_v, sg, bufs, gsems,
                    wsems):
    sc = lax.axis_index("c")
    sub = lax.axis_index("s")
    lo = (sc * _NUM_SUBCORES + sub) * rpt
    lanes = jax.lax.iota(jnp.int32, _L)
    lane0 = lanes == 0
    ones = jnp.ones((_L,), jnp.int32)

    pltpu.sync_copy(idx_hbm, idx_full)

    for j in range(rpt // _L):
      hist[pl.ds(j * _L, _L)] = jnp.zeros((_L,), jnp.int32)

    def scan_body(i, off):
      pos = i * _L + lanes
      v = idx_full[pl.ds(i * _L, _L)]
      m = (v >= lo) & (v < lo + rpt)
      c = plsc.cumsum(m.astype(jnp.int32))
      dst = off + c - 1
      plsc.store_scatter(wl_pos, [dst], pos, mask=m)
      plsc.store_scatter(wl_row, [dst], v - lo, mask=m)
      plsc.addupdate_scatter(hist, [v - lo], ones, mask=m)
      return off + jnp.max(c)

    k = lax.fori_loop(0, n_out // _L, scan_body, jnp.int32(0))

    carry = jnp.int32(0)
    for j in range(rpt // _L):
      h = hist[pl.ds(j * _L, _L)]
      cs = plsc.cumsum(h)
      excl = cs - h + carry
      starts[pl.ds(j * _L, _L)] = excl
      cursor[pl.ds(j * _L, _L)] = excl
      carry = carry + jnp.max(cs)

    def group_body(e, carry2):
      ee = jnp.full((_L,), e, jnp.int32)
      p = plsc.load_gather(wl_pos, [ee])
      r = plsc.load_gather(wl_row, [ee])
      slot = plsc.load_gather(cursor, [r])
      plsc.store_scatter(grouped, [slot], p, mask=lane0)
      plsc.addupdate_scatter(cursor, [r], ones, mask=lane0)
      return carry2

    lax.fori_loop(0, k, group_body, 0)

    for j in range(rpt // _L):
      hist_pub[pl.ds(j * _L, _L)] = hist[pl.ds(j * _L, _L)]
    hist_pub[pl.ds(rpt, _L)] = jnp.full((_L,), k, jnp.int32)
    for j in range(rpt // _L + 1, _C // _L):
      hist_pub[pl.ds(j * _L, _L)] = jnp.zeros((_L,), jnp.int32)
    pltpu.sync_copy(hist_pub, hist_hbm.at[sc, sub])
    plsc.subcore_barrier()

    pltpu.sync_copy(hist_hbm.at[sc], hist_all_v)
    kvals = plsc.load_gather(hist_all_v, [lanes, jnp.full((_L,), rpt,
                                                          jnp.int32)])
    kpad = ((kvals + (_C - 1)) // _C) * _C
    my_off = jnp.sum(jnp.where(lanes < sub, kpad, 0))

    k_sc = jnp.sum(kvals)
    lo_t = sub * k_sc // _NUM_SUBCORES
    hi_t = (sub + 1) * k_sc // _NUM_SUBCORES
    g_lo = jnp.int32(0)
    g_hi = jnp.int32(0)
    for t in range(_NUM_SUBCORES):
      base_pad = jnp.sum(jnp.where(lanes < t, kpad, 0))
      base_unp = jnp.sum(jnp.where(lanes < t, kvals, 0))
      carry4 = jnp.int32(0)
      for j in range(rpt // _L):
        h = hist_all_v[t, pl.ds(j * _L, _L)]
        cs = plsc.cumsum(h)
        excl = cs - h + carry4
        sg[pl.ds(t * rpt + j * _L, _L)] = excl + base_pad
        s_unp = excl + base_unp
        g_lo = g_lo + jnp.sum((s_unp < lo_t).astype(jnp.int32))
        g_hi = g_hi + jnp.sum((s_unp < hi_t).astype(jnp.int32))
        carry4 = carry4 + jnp.max(cs)

    n_mine0 = g_hi - g_lo

    def prime_meta(u_rel):
      u = jnp.minimum(g_lo + u_rel, rpc - 1)
      uu = jnp.full((_L,), u, jnp.int32)
      cnt = jnp.max(plsc.load_gather(hist_all_v, [uu // rpt, uu % rpt]))
      return jnp.where(u_rel < n_mine0, cnt, 0), u

    for i in range(nbuf):
      cnt_i, u_i = prime_meta(jnp.int32(i))

      @pl.when(cnt_i > 0)
      def _():
        pltpu.async_copy(table_hbm.at[pl.ds(sc * rpc + u_i, 1)], bufs[i],
                         gsems[i])

    def pub_body(c2, carry3):
      pltpu.sync_copy(grouped.at[pl.ds(c2 * _C, _C)],
                      grouped_hbm.at[sc].at[
                          pl.ds(pl.multiple_of(my_off + c2 * _C, _C), _C)])
      return carry3

    lax.fori_loop(0, (k + _C - 1) // _C, pub_body, 0)
    plsc.subcore_barrier()

    pltpu.sync_copy(grouped_hbm.at[sc], grouped_all)

    def row_meta(u):
      uu = jnp.full((_L,), u, jnp.int32)
      cnt = jnp.max(
          plsc.load_gather(hist_all_v, [uu // rpt, uu % rpt]))
      st = jnp.max(plsc.load_gather(sg, [uu]))
      return cnt, st

    n_mine = g_hi - g_lo

    def active_cnt(u_rel):
      u = jnp.minimum(g_lo + u_rel, rpc - 1)
      cnt, st = row_meta(u)
      cnt = jnp.where(u_rel < n_mine, cnt, 0)
      return cnt, st, u

    def start_gather(u, b):
      pltpu.async_copy(table_hbm.at[pl.ds(sc * rpc + u, 1)], bufs[b],
                       gsems[b])

    def wait_gather(u, b):
      pltpu.make_async_copy(table_hbm.at[pl.ds(sc * rpc + u, 1)], bufs[b],
                            gsems[b]).wait()

    def maybe_start_gather(u_rel, b):
      cnt, _, u = active_cnt(u_rel)

      @pl.when((cnt > 0) & (u_rel >= nbuf))
      def _():
        start_gather(u, b)

    def fire_writes(u_rel, b):
      cnt, st, u = active_cnt(u_rel)

      @pl.when(cnt > 0)
      def _():
        wait_gather(u, b)

      def wbody(j, carry5):
        gi = jnp.clip(st + j, 0, pad_cap - 1)
        pos = jnp.max(
            plsc.load_gather(grouped_all, [jnp.full((_L,), gi, jnp.int32)]))
        pos = jnp.clip(pos, 0, n_out - 1)
        pltpu.async_copy(bufs[b], out_hbm.at[pl.ds(pos, 1)], wsems[b])
        return carry5

      lax.fori_loop(0, cnt, wbody, 0)
      return cnt

    def drain_writes(cnt, b):
      def dbody(j, carry6):
        pltpu.make_async_copy(bufs[b], out_hbm.at[pl.ds(0, 1)],
                              wsems[b]).wait()
        return carry6

      lax.fori_loop(0, cnt, dbody, 0)

    def slot(u_rel, b, pending):
      cnt = fire_writes(u_rel, b)
      bd = (b + 2) % nbuf
      drain_writes(pending[0], bd)
      maybe_start_gather(u_rel + 2, bd)
      return pending[1:] + (cnt,)

    def round_body(jr, pending):
      for b in range(nbuf):
        pending = slot(jr * nbuf + b, b, pending)
      return pending

    pending = tuple(jnp.int32(0) for _ in range(nbuf - 2))
    pending = lax.fori_loop(0, (n_mine + nbuf - 1) // nbuf, round_body,
                            pending)
    for t in range(nbuf - 2):
      drain_writes(pending[t], (t + 2) % nbuf)

  return gather_kernel


def kernel(prefix, emb_table):
  b, s = prefix.shape
  v, d = emb_table.shape
  n = b * s
  out, _, _ = _make_gather(n, v, d, 5)(prefix.reshape(n), emb_table)
  return out.reshape(b, s, d)


# speedup vs baseline: 1.0059x; 1.0059x over previous
import functools

import jax
import jax.numpy as jnp
from jax import lax
from jax.experimental import pallas as pl
from jax.experimental.pallas import tpu as pltpu
from jax.experimental.pallas import tpu_sc as plsc

_NUM_CORES = 2
_NUM_SUBCORES = 16
_NUM_WORKERS = _NUM_CORES * _NUM_SUBCORES
_L = 16


def _make_gather(n_out: int, n_rows: int, d: int, nbuf: int):
  rpt = n_rows // _NUM_WORKERS
  rpc = n_rows // _NUM_CORES
  _C = 128
  pad_cap = n_out + _NUM_SUBCORES * _C
  mesh = plsc.VectorSubcoreMesh(core_axis_name="c", subcore_axis_name="s")

  @functools.partial(
      pl.kernel,
      out_type=[
          jax.ShapeDtypeStruct((n_out, d), jnp.float32),
          jax.ShapeDtypeStruct((_NUM_CORES, _NUM_SUBCORES, _C), jnp.int32),
          jax.ShapeDtypeStruct((_NUM_CORES, pad_cap), jnp.int32),
      ],
      mesh=mesh,
      compiler_params=pltpu.CompilerParams(needs_layout_passes=False),
      scratch_types=[
          pltpu.VMEM((n_out,), jnp.int32),
          pltpu.VMEM((n_out,), jnp.int32),
          pltpu.VMEM((n_out,), jnp.int32),
          pltpu.VMEM((n_out,), jnp.int32),
          pltpu.VMEM((pad_cap,), jnp.int32),
          pltpu.VMEM((rpt,), jnp.int32),
          pltpu.VMEM((rpt,), jnp.int32),
          pltpu.VMEM((rpt,), jnp.int32),
          pltpu.VMEM((_C,), jnp.int32),
          pltpu.VMEM((_NUM_SUBCORES, _C), jnp.int32),
          pltpu.VMEM((rpc,), jnp.int32),
          [pltpu.VMEM((1, d), jnp.float32) for _ in range(nbuf)],
          [pltpu.SemaphoreType.DMA for _ in range(nbuf)],
          [pltpu.SemaphoreType.DMA for _ in range(nbuf)],
      ],
  )
  def gather_kernel(idx_hbm, table_hbm, out_hbm, hist_hbm, grouped_hbm,
                    idx_full, wl_pos, wl_row, grouped, grouped_all, hist,
                    starts, cursor, hist_pub, hist_all_v, sg, bufs, gsems,
                    wsems):
    sc = lax.axis_index("c")
    sub = lax.axis_index("s")
    lo = (sc * _NUM_SUBCORES + sub) * rpt
    lanes = jax.lax.iota(jnp.int32, _L)
    lane0 = lanes == 0
    ones = jnp.ones((_L,), jnp.int32)

    pltpu.sync_copy(idx_hbm, idx_full)

    for j in range(rpt // _L):
      hist[pl.ds(j * _L, _L)] = jnp.zeros((_L,), jnp.int32)

    def scan_body(i, off_v):
      pos = i * _L + lanes
      v = idx_full[pl.ds(i * _L, _L)]
      m = (v >= lo) & (v < lo + rpt)
      c = plsc.cumsum(m.astype(jnp.int32))
      dst = off_v + c - 1
      plsc.store_scatter(wl_pos, [dst], pos, mask=m)
      plsc.store_scatter(wl_row, [dst], v - lo, mask=m)
      plsc.addupdate_scatter(hist, [v - lo], ones, mask=m)
      return off_v + plsc.all_reduce_population_count(m)

    off_v = lax.fori_loop(0, n_out // _L, scan_body,
                          jnp.zeros((_L,), jnp.int32))
    k = jnp.max(off_v)

    carry = jnp.int32(0)
    for j in range(rpt // _L):
      h = hist[pl.ds(j * _L, _L)]
      cs = plsc.cumsum(h)
      excl = cs - h + carry
      starts[pl.ds(j * _L, _L)] = excl
      cursor[pl.ds(j * _L, _L)] = excl
      carry = carry + jnp.max(cs)

    def group_body(e, carry2):
      ee = jnp.full((_L,), e, jnp.int32)
      p = plsc.load_gather(wl_pos, [ee])
      r = plsc.load_gather(wl_row, [ee])
      slot = plsc.load_gather(cursor, [r])
      plsc.store_scatter(grouped, [slot], p, mask=lane0)
      plsc.addupdate_scatter(cursor, [r], ones, mask=lane0)
      return carry2

    lax.fori_loop(0, k, group_body, 0)

    for j in range(rpt // _L):
      hist_pub[pl.ds(j * _L, _L)] = hist[pl.ds(j * _L, _L)]
    hist_pub[pl.ds(rpt, _L)] = jnp.full((_L,), k, jnp.int32)
    for j in range(rpt // _L + 1, _C // _L):
      hist_pub[pl.ds(j * _L, _L)] = jnp.zeros((_L,), jnp.int32)
    pltpu.sync_copy(hist_pub, hist_hbm.at[sc, sub])
    plsc.subcore_barrier()

    pltpu.sync_copy(hist_hbm.at[sc], hist_all_v)
    kvals = plsc.load_gather(hist_all_v, [lanes, jnp.full((_L,), rpt,
                                                          jnp.int32)])
    kpad = ((kvals + (_C - 1)) // _C) * _C
    my_off = jnp.sum(jnp.where(lanes < sub, kpad, 0))

    def pub_body(c2, carry3):
      pltpu.sync_copy(grouped.at[pl.ds(c2 * _C, _C)],
                      grouped_hbm.at[sc].at[
                          pl.ds(pl.multiple_of(my_off + c2 * _C, _C), _C)])
      return carry3

    lax.fori_loop(0, (k + _C - 1) // _C, pub_body, 0)

    k_sc = jnp.sum(kvals)
    lo_t = sub * k_sc // _NUM_SUBCORES
    hi_t = (sub + 1) * k_sc // _NUM_SUBCORES
    g_lo = jnp.int32(0)
    g_hi = jnp.int32(0)
    for t in range(_NUM_SUBCORES):
      base_pad = jnp.sum(jnp.where(lanes < t, kpad, 0))
      base_unp = jnp.sum(jnp.where(lanes < t, kvals, 0))
      carry4 = jnp.int32(0)
      for j in range(rpt // _L):
        h = hist_all_v[t, pl.ds(j * _L, _L)]
        cs = plsc.cumsum(h)
        excl = cs - h + carry4
        sg[pl.ds(t * rpt + j * _L, _L)] = excl + base_pad
        s_unp = excl + base_unp
        g_lo = g_lo + jnp.sum((s_unp < lo_t).astype(jnp.int32))
        g_hi = g_hi + jnp.sum((s_unp < hi_t).astype(jnp.int32))
        carry4 = carry4 + jnp.max(cs)

    n_mine0 = g_hi - g_lo

    def prime_meta(u_rel):
      u = jnp.minimum(g_lo + u_rel, rpc - 1)
      uu = jnp.full((_L,), u, jnp.int32)
      cnt = jnp.max(plsc.load_gather(hist_all_v, [uu // rpt, uu % rpt]))
      return jnp.where(u_rel < n_mine0, cnt, 0), u

    for i in range(nbuf):
      cnt_i, u_i = prime_meta(jnp.int32(i))

      @pl.when(cnt_i > 0)
      def _():
        pltpu.async_copy(table_hbm.at[pl.ds(sc * rpc + u_i, 1)], bufs[i],
                         gsems[i])

    plsc.subcore_barrier()
    pltpu.sync_copy(grouped_hbm.at[sc], grouped_all)

    def row_meta(u):
      uu = jnp.full((_L,), u, jnp.int32)
      cnt = jnp.max(
          plsc.load_gather(hist_all_v, [uu // rpt, uu % rpt]))
      st = jnp.max(plsc.load_gather(sg, [uu]))
      return cnt, st

    n_mine = g_hi - g_lo

    def active_cnt(u_rel):
      u = jnp.minimum(g_lo + u_rel, rpc - 1)
      cnt, st = row_meta(u)
      cnt = jnp.where(u_rel < n_mine, cnt, 0)
      return cnt, st, u

    def start_gather(u, b):
      pltpu.async_copy(table_hbm.at[pl.ds(sc * rpc + u, 1)], bufs[b],
                       gsems[b])

    def wait_gather(u, b):
      pltpu.make_async_copy(table_hbm.at[pl.ds(sc * rpc + u, 1)], bufs[b],
                            gsems[b]).wait()

    def maybe_start_gather(u_rel, b):
      cnt, _, u = active_cnt(u_rel)

      @pl.when((cnt > 0) & (u_rel >= nbuf))
      def _():
        start_gather(u, b)

    def fire_writes(u_rel, b):
      cnt, st, u = active_cnt(u_rel)

      @pl.when(cnt > 0)
      def _():
        wait_gather(u, b)

      def wbody(j, carry5):
        gi = jnp.clip(st + j, 0, pad_cap - 1)
        pos = jnp.max(
            plsc.load_gather(grouped_all, [jnp.full((_L,), gi, jnp.int32)]))
        pos = jnp.clip(pos, 0, n_out - 1)
        pltpu.async_copy(bufs[b], out_hbm.at[pl.ds(pos, 1)], wsems[b])
        return carry5

      lax.fori_loop(0, cnt, wbody, 0)
      return cnt

    def drain_writes(cnt, b):
      def dbody(j, carry6):
        pltpu.make_async_copy(bufs[b], out_hbm.at[pl.ds(0, 1)],
                              wsems[b]).wait()
        return carry6

      lax.fori_loop(0, cnt, dbody, 0)

    def slot(u_rel, b, pending):
      cnt = fire_writes(u_rel, b)
      bd = (b + 2) % nbuf
      drain_writes(pending[0], bd)
      maybe_start_gather(u_rel + 2, bd)
      return pending[1:] + (cnt,)

    def round_body(jr, pending):
      for b in range(nbuf):
        pending = slot(jr * nbuf + b, b, pending)
      return pending

    pending = tuple(jnp.int32(0) for _ in range(nbuf - 2))
    pending = lax.fori_loop(0, (n_mine + nbuf - 1) // nbuf, round_body,
                            pending)
    for t in range(nbuf - 2):
      drain_writes(pending[t], (t + 2) % nbuf)

  return gather_kernel


def kernel(prefix, emb_table):
  b, s = prefix.shape
  v, d = emb_table.shape
  n = b * s
  out, _, _ = _make_gather(n, v, d, 5)(prefix.reshape(n), emb_table)
  return out.reshape(b, s, d)
